# Initial kernel scaffold; baseline (speedup 1.0000x reference)
import functools

import jax
import jax.numpy as jnp
from jax import lax
from jax.experimental import pallas as pl
from jax.experimental.pallas import tpu as pltpu
from jax.experimental.pallas import tpu_sc as plsc

N = 10000
IN_F = 128
HID = 32
HEADS = 4
NCLS = 16
E = 320000

NC = 2
NS = 16
NW = NC * NS
EPW = E // NW
NPS = N // NS
IDXW = 100

_F32 = jnp.float32


def _proj1_body(feat_ref, w_ref, b_ref, h_ref, elrt_ref):
    h = jnp.dot(feat_ref[...], w_ref[...], preferred_element_type=_F32)
    h_ref[...] = h
    elrt_ref[...] = lax.dot_general(
        b_ref[...], h, (((0,), (1,)), ((), ())), preferred_element_type=_F32)


def _proj1(feat, w1, b1mat):
    blk = 1000
    return pl.pallas_call(
        _proj1_body,
        grid=(N // blk,),
        in_specs=[
            pl.BlockSpec((blk, IN_F), lambda i: (i, 0)),
            pl.BlockSpec((IN_F, IN_F), lambda i: (0, 0)),
            pl.BlockSpec((IN_F, 2 * HEADS), lambda i: (0, 0)),
        ],
        out_specs=[
            pl.BlockSpec((blk, IN_F), lambda i: (i, 0)),
            pl.BlockSpec((2 * HEADS, blk), lambda i: (0, i)),
        ],
        out_shape=[
            jax.ShapeDtypeStruct((N, IN_F), _F32),
            jax.ShapeDtypeStruct((2 * HEADS, N), _F32),
        ],
    )(feat, w1, b1mat)


def _combine_body(u_ref, es_ref, p_ref, b1_ref, w2_ref, b2m_ref,
                  h2_ref, elrt_ref):
    es = jnp.sum(es_ref[...], axis=0)
    inv = 1.0 / (es + 1e-9)
    inv_exp = lax.dot_general(
        inv, p_ref[...], (((0,), (0,)), ((), ())), preferred_element_type=_F32)
    u = u_ref[0] + u_ref[1]
    o1 = jnp.maximum(u * inv_exp + b1_ref[...], 0.0)
    h2 = jnp.dot(o1, w2_ref[...], preferred_element_type=_F32)
    h2_ref[...] = h2
    elrt_ref[...] = lax.dot_general(
        b2m_ref[...], h2, (((0,), (1,)), ((), ())), preferred_element_type=_F32)


def _combine(u_parts, esum_parts, p1, b1flat, w2, b2mat):
    blk = 1000
    return pl.pallas_call(
        _combine_body,
        grid=(N // blk,),
        in_specs=[
            pl.BlockSpec((NC, blk, IN_F), lambda i: (0, i, 0)),
            pl.BlockSpec((NW, HEADS, blk), lambda i: (0, 0, i)),
            pl.BlockSpec((HEADS, IN_F), lambda i: (0, 0)),
            pl.BlockSpec((1, IN_F), lambda i: (0, 0)),
            pl.BlockSpec((IN_F, NCLS), lambda i: (0, 0)),
            pl.BlockSpec((NCLS, 2), lambda i: (0, 0)),
        ],
        out_specs=[
            pl.BlockSpec((blk, NCLS), lambda i: (i, 0)),
            pl.BlockSpec((2, blk), lambda i: (0, i)),
        ],
        out_shape=[
            jax.ShapeDtypeStruct((N, NCLS), _F32),
            jax.ShapeDtypeStruct((2, N), _F32),
        ],
    )(u_parts, esum_parts, p1, b1flat, w2, b2mat)


def _final_body(u_ref, es_ref, p_ref, b2_ref, out_ref):
    es = jnp.sum(es_ref[...], axis=0)
    inv = 1.0 / (es + 1e-9)
    inv_exp = lax.dot_general(
        inv, p_ref[...], (((0,), (0,)), ((), ())), preferred_element_type=_F32)
    u = u_ref[0] + u_ref[1]
    out_ref[...] = u * inv_exp + b2_ref[...]


def _final(u_parts, esum_parts, p2, b2flat):
    blk = 1000
    return pl.pallas_call(
        _final_body,
        grid=(N // blk,),
        in_specs=[
            pl.BlockSpec((NC, blk, NCLS), lambda i: (0, i, 0)),
            pl.BlockSpec((NW, 1, blk), lambda i: (0, 0, i)),
            pl.BlockSpec((1, NCLS), lambda i: (0, 0)),
            pl.BlockSpec((1, NCLS), lambda i: (0, 0)),
        ],
        out_specs=pl.BlockSpec((blk, NCLS), lambda i: (i, 0)),
        out_shape=jax.ShapeDtypeStruct((N, NCLS), _F32),
    )(u_parts, esum_parts, p2, b2flat)


def _make_pass1(h_heads):
    tbl_sz = 2 * h_heads * N
    acc_sz = h_heads * N
    chunk = 400
    mesh = plsc.VectorSubcoreMesh(core_axis_name="c", subcore_axis_name="s")

    @functools.partial(
        pl.kernel,
        out_type=(
            jax.ShapeDtypeStruct((h_heads, E), _F32),
            jax.ShapeDtypeStruct((NW, acc_sz), _F32),
        ),
        mesh=mesh,
        scratch_types=[
            pltpu.VMEM((tbl_sz,), _F32),
            pltpu.VMEM((acc_sz,), _F32),
            pltpu.VMEM((chunk,), jnp.int32),
            pltpu.VMEM((chunk,), jnp.int32),
            pltpu.VMEM((h_heads, chunk), _F32),
        ],
    )
    def pass1(elr_hbm, src_hbm, dst_hbm, ee_hbm, esum_hbm,
              tbl, acc, srcb, dstb, eeb):
        cid = lax.axis_index("c")
        sid = lax.axis_index("s")
        wid = sid * NC + cid
        pltpu.sync_copy(elr_hbm, tbl)
        zeros = jnp.zeros((16,), _F32)

        @pl.loop(0, acc_sz, step=16)
        def _zero(i):
            acc[pl.ds(i, 16)] = zeros

        base = wid * EPW

        @pl.loop(0, EPW, step=chunk)
        def _chunk(off):
            start = base + off
            pltpu.sync_copy(src_hbm.at[pl.ds(start, chunk)], srcb)
            pltpu.sync_copy(dst_hbm.at[pl.ds(start, chunk)], dstb)

            @pl.loop(0, chunk, step=16)
            def _grp(g):
                sv = srcb[pl.ds(g, 16)]
                dv = dstb[pl.ds(g, 16)]
                for k in range(h_heads):
                    el = plsc.load_gather(tbl, [sv + (k * N)])
                    er = plsc.load_gather(tbl, [dv + ((h_heads + k) * N)])
                    e = el + er
                    e = jnp.maximum(e, 0.2 * e)
                    ee = jnp.exp(e)
                    eeb[k, pl.ds(g, 16)] = ee
                    plsc.addupdate_scatter(acc, [dv + (k * N)], ee)

            pltpu.sync_copy(eeb, ee_hbm.at[:, pl.ds(start, chunk)])

        pltpu.sync_copy(acc, esum_hbm.at[wid])

    return pass1


def _make_pass2(h_heads, feats):
    d_model = h_heads * feats
    chunk = 200
    sub = chunk // IDXW
    mesh = plsc.VectorSubcoreMesh(core_axis_name="c", subcore_axis_name="s")

    @functools.partial(
        pl.kernel,
        out_type=jax.ShapeDtypeStruct((NC, N, d_model), _F32),
        mesh=mesh,
        scratch_types=[
            pltpu.VMEM((sub, IDXW), jnp.int32),
            pltpu.VMEM((sub, IDXW), jnp.int32),
            pltpu.VMEM((h_heads, chunk), _F32),
            pltpu.VMEM((chunk, d_model), _F32),
            pltpu.VMEM_SHARED((N, d_model), _F32),
        ],
    )
    def pass2(h_hbm, src2_hbm, dst2_hbm, ee_hbm, u_hbm,
              srcb, dstb, eeb, rows, u_sh):
        cid = lax.axis_index("c")
        sid = lax.axis_index("s")
        wid = sid * NC + cid
        zeros = jnp.zeros((16,), _F32)

        @pl.loop(0, chunk)
        def _zr(r):
            for jj in range(d_model // 16):
                rows[r, pl.ds(jj * 16, 16)] = zeros

        nb = sid * NPS
        full = NPS // chunk
        rem = NPS - full * chunk
        for t in range(full):
            pltpu.sync_copy(rows, u_sh.at[pl.ds(nb + t * chunk, chunk)])
        if rem:
            pltpu.sync_copy(rows.at[pl.ds(0, rem)],
                            u_sh.at[pl.ds(nb + full * chunk, rem)])
        plsc.subcore_barrier()

        base = wid * EPW

        @pl.loop(0, EPW, step=chunk)
        def _chunk(off):
            start = base + off
            row0 = (wid * (EPW // IDXW)) + (off // IDXW)
            pltpu.sync_copy(src2_hbm.at[pl.ds(row0, sub)], srcb)
            pltpu.sync_copy(dst2_hbm.at[pl.ds(row0, sub)], dstb)
            pltpu.sync_copy(ee_hbm.at[:, pl.ds(start, chunk)], eeb)
            for j in range(sub):
                pltpu.sync_copy(h_hbm.at[srcb.at[j]],
                                rows.at[pl.ds(j * IDXW, IDXW)])

            @pl.loop(0, chunk)
            def _scale(e):
                for jj in range(d_model // 16):
                    k = (jj * 16) // feats
                    s = eeb[k, e]
                    rows[e, pl.ds(jj * 16, 16)] = (
                        rows[e, pl.ds(jj * 16, 16)] * s)

            for j in range(sub):
                pltpu.sync_copy(rows.at[pl.ds(j * IDXW, IDXW)],
                                u_sh.at[dstb.at[j]], add=True)

        plsc.subcore_barrier()
        pltpu.sync_copy(u_sh.at[pl.ds(nb, NPS)],
                        u_hbm.at[cid, pl.ds(nb, NPS)])

    return pass2


_pass1_l1 = _make_pass1(HEADS)
_pass1_l2 = _make_pass1(1)
_pass2_l1 = _make_pass2(HEADS, HID)
_pass2_l2 = _make_pass2(1, NCLS)


def _attn_mat(attn, in_dim):
    h, f = attn.shape
    mask = (jnp.arange(in_dim)[:, None] // f) == jnp.arange(h)[None, :]
    return jnp.where(mask, attn.reshape(-1)[:, None], 0.0).astype(_F32)


def kernel(feat, edge_index, W1, al1, ar1, b1, W2, al2, ar2, b2):
    src = edge_index[0].astype(jnp.int32)
    dst = edge_index[1].astype(jnp.int32)
    src2 = src.reshape(E // IDXW, IDXW)
    dst2 = dst.reshape(E // IDXW, IDXW)

    b1mat = jnp.concatenate(
        [_attn_mat(al1, IN_F), _attn_mat(ar1, IN_F)], axis=1)
    b2mat = jnp.concatenate(
        [_attn_mat(al2, NCLS), _attn_mat(ar2, NCLS)], axis=1)
    p1 = ((jnp.arange(IN_F)[None, :] // HID)
          == jnp.arange(HEADS)[:, None]).astype(_F32)
    p2 = jnp.ones((1, NCLS), _F32)

    h1, elr1t = _proj1(feat, W1, b1mat)
    ee1, esum1 = _pass1_l1(elr1t.reshape(-1), src, dst)
    u1 = _pass2_l1(h1, src2, dst2, ee1)
    h2, elr2t = _combine(u1, esum1.reshape(NW, HEADS, N), p1,
                         b1.reshape(1, -1), W2, b2mat)
    ee2, esum2 = _pass1_l2(elr2t.reshape(-1), src, dst)
    u2 = _pass2_l2(h2, src2, dst2, ee2)
    out = _final(u2, esum2.reshape(NW, 1, N), p2, b2.reshape(1, -1))
    return out


# speedup vs baseline: 29.6682x; 29.6682x over previous
import dataclasses
import functools

import jax
import jax.numpy as jnp
from jax import lax
from jax.experimental import pallas as pl
from jax.experimental.pallas import tpu as pltpu
from jax.experimental.pallas import tpu_sc as plsc

N = 10000
IN_F = 128
HID = 32
HEADS = 4
NCLS = 16
E = 320000

NC = 2
NS = 16
NW = NC * NS
EPW = E // NW

_F32 = jnp.float32

_SC_CP = pltpu.CompilerParams()
if "needs_layout_passes" in pltpu.CompilerParams.__dataclass_fields__:
    _SC_CP = dataclasses.replace(_SC_CP, needs_layout_passes=False)


def _proj1_body(feat_ref, w_ref, b_ref, h_ref, elr_ref):
    h = jnp.dot(feat_ref[...], w_ref[...], preferred_element_type=_F32)
    h_ref[...] = h
    elr_ref[...] = jnp.dot(h, b_ref[...], preferred_element_type=_F32)


def _proj1(feat, w1, b1mat):
    blk = 1000
    return pl.pallas_call(
        _proj1_body,
        grid=(N // blk,),
        in_specs=[
            pl.BlockSpec((blk, IN_F), lambda i: (i, 0)),
            pl.BlockSpec((IN_F, IN_F), lambda i: (0, 0)),
            pl.BlockSpec((IN_F, 2 * HEADS), lambda i: (0, 0)),
        ],
        out_specs=[
            pl.BlockSpec((blk, IN_F), lambda i: (i, 0)),
            pl.BlockSpec((blk, 2 * HEADS), lambda i: (i, 0)),
        ],
        out_shape=[
            jax.ShapeDtypeStruct((N, IN_F), _F32),
            jax.ShapeDtypeStruct((N, 2 * HEADS), _F32),
        ],
    )(feat, w1, b1mat)


def _inv_body(es_ref, inv_ref):
    inv_ref[...] = 1.0 / (jnp.sum(es_ref[...], axis=0) + 1e-9)


def _inv_reduce(esum_parts):
    m = esum_parts.shape[1]
    return pl.pallas_call(
        _inv_body,
        in_specs=[pl.BlockSpec((NW, m), lambda: (0, 0))],
        out_specs=pl.BlockSpec((m,), lambda: (0,)),
        out_shape=jax.ShapeDtypeStruct((m,), _F32),
    )(esum_parts)


def _combine_body(u_ref, inv_ref, p_ref, b1_ref, w2_ref, b2m_ref,
                  h2_ref, elr_ref):
    inv_exp = jnp.dot(inv_ref[...], p_ref[...],
                      preferred_element_type=_F32)
    u = u_ref[0] + u_ref[1]
    o1 = jnp.maximum(u * inv_exp + b1_ref[...], 0.0)
    h2 = jnp.dot(o1, w2_ref[...], preferred_element_type=_F32)
    h2_ref[...] = h2
    elr_ref[...] = jnp.dot(h2, b2m_ref[...], preferred_element_type=_F32)


def _combine(u_parts, inv1, p1, b1flat, w2, b2mat):
    blk = 1000
    return pl.pallas_call(
        _combine_body,
        grid=(N // blk,),
        in_specs=[
            pl.BlockSpec((NC, blk, IN_F), lambda i: (0, i, 0)),
            pl.BlockSpec((blk, HEADS), lambda i: (i, 0)),
            pl.BlockSpec((HEADS, IN_F), lambda i: (0, 0)),
            pl.BlockSpec((1, IN_F), lambda i: (0, 0)),
            pl.BlockSpec((IN_F, IN_F), lambda i: (0, 0)),
            pl.BlockSpec((IN_F, 2), lambda i: (0, 0)),
        ],
        out_specs=[
            pl.BlockSpec((blk, IN_F), lambda i: (i, 0)),
            pl.BlockSpec((blk, 2), lambda i: (i, 0)),
        ],
        out_shape=[
            jax.ShapeDtypeStruct((N, IN_F), _F32),
            jax.ShapeDtypeStruct((N, 2), _F32),
        ],
    )(u_parts, inv1, p1, b1flat, w2, b2mat)


def _final_body(u_ref, inv_ref, p_ref, b2_ref, out_ref):
    inv_exp = jnp.dot(inv_ref[...], p_ref[...],
                      preferred_element_type=_F32)
    u = u_ref[0][:, :NCLS] + u_ref[1][:, :NCLS]
    out_ref[...] = u * inv_exp + b2_ref[...]


def _final(u_parts, inv2, p2, b2flat):
    blk = 1000
    return pl.pallas_call(
        _final_body,
        grid=(N // blk,),
        in_specs=[
            pl.BlockSpec((NC, blk, IN_F), lambda i: (0, i, 0)),
            pl.BlockSpec((blk, 1), lambda i: (i, 0)),
            pl.BlockSpec((1, NCLS), lambda i: (0, 0)),
            pl.BlockSpec((1, NCLS), lambda i: (0, 0)),
        ],
        out_specs=pl.BlockSpec((blk, NCLS), lambda i: (i, 0)),
        out_shape=jax.ShapeDtypeStruct((N, NCLS), _F32),
    )(u_parts, inv2, p2, b2flat)


def _make_pass1(h_heads):
    tbl_sz = 2 * h_heads * N
    acc_sz = h_heads * N
    ch = 640
    nck = -(-E // ch // NW)
    mesh = plsc.VectorSubcoreMesh(core_axis_name="c", subcore_axis_name="s",
                                  num_cores=NC, num_subcores=NS)

    @functools.partial(
        pl.kernel,
        out_type=(
            jax.ShapeDtypeStruct((h_heads * E,), _F32),
            jax.ShapeDtypeStruct((NW, 1, acc_sz), _F32),
        ),
        mesh=mesh,
        compiler_params=_SC_CP,
        scratch_types=[
            pltpu.VMEM((tbl_sz,), _F32),
            pltpu.VMEM((acc_sz,), _F32),
            pltpu.VMEM((ch,), jnp.int32),
            pltpu.VMEM((ch,), jnp.int32),
            pltpu.VMEM((h_heads * ch,), _F32),
        ],
    )
    def pass1(elr_hbm, src_hbm, dst_hbm, ee_hbm, esum_hbm,
              tbl, acc, srcb, dstb, eeb):
        cid = lax.axis_index("c")
        sid = lax.axis_index("s")
        wid = sid * NC + cid
        pltpu.sync_copy(elr_hbm, tbl)
        zeros = jnp.zeros((16,), _F32)

        @pl.loop(0, acc_sz, step=16)
        def _zero(i):
            acc[pl.ds(i, 16)] = zeros

        @pl.loop(0, nck)
        def _chunk(ci):
            start = (ci * NW + wid) * ch

            @pl.when(start < E)
            def _do():
                pltpu.sync_copy(src_hbm.at[pl.ds(start, ch)], srcb)
                pltpu.sync_copy(dst_hbm.at[pl.ds(start, ch)], dstb)

                @pl.loop(0, ch, step=16)
                def _grp(g):
                    sv = srcb[pl.ds(g, 16)]
                    dv = dstb[pl.ds(g, 16)]
                    sv2 = sv * (2 * h_heads)
                    dv2 = dv * (2 * h_heads)
                    dva = dv * h_heads
                    for k in range(h_heads):
                        el = plsc.load_gather(tbl, [sv2 + k])
                        er = plsc.load_gather(tbl, [dv2 + (h_heads + k)])
                        e = el + er
                        e = jnp.maximum(e, 0.2 * e)
                        ee = jnp.exp(e)
                        eeb[pl.ds(k * ch + g, 16)] = ee
                        plsc.addupdate_scatter(acc, [dva + k], ee)

                for k in range(h_heads):
                    pltpu.sync_copy(eeb.at[pl.ds(k * ch, ch)],
                                    ee_hbm.at[pl.ds(k * E + start, ch)])

        pltpu.sync_copy(acc, esum_hbm.at[wid, 0])

    return pass1


def _make_pass2(h_heads, feats):
    d_model = h_heads * feats
    ch = 128
    nck = -(-E // ch // NW)
    nrk = N // ch
    rtail = N - nrk * ch
    mesh = plsc.VectorSubcoreMesh(core_axis_name="c", subcore_axis_name="s",
                                  num_cores=NC, num_subcores=NS)

    @functools.partial(
        pl.kernel,
        out_type=jax.ShapeDtypeStruct((NC, N, d_model), _F32),
        mesh=mesh,
        compiler_params=_SC_CP,
        scratch_types=[
            pltpu.VMEM((ch,), jnp.int32),
            pltpu.VMEM((ch,), jnp.int32),
            pltpu.VMEM((h_heads * ch,), _F32),
            pltpu.VMEM((ch, d_model), _F32),
            pltpu.VMEM_SHARED((N, d_model), _F32),
        ],
    )
    def pass2(h_hbm, src_hbm, dst_hbm, ee_hbm, u_hbm,
              srcb, dstb, eeb, rows, u_sh):
        cid = lax.axis_index("c")
        sid = lax.axis_index("s")
        wid = sid * NC + cid
        zeros = jnp.zeros((16,), _F32)

        @pl.loop(0, ch)
        def _zr(r):
            for jj in range(d_model // 16):
                rows[r, pl.ds(jj * 16, 16)] = zeros

        @pl.loop(0, nrk)
        def _zs(bi):
            @pl.when(bi % NS == sid)
            def _():
                pltpu.sync_copy(rows, u_sh.at[pl.ds(bi * ch, ch)])

        @pl.when(sid == 0)
        def _zt():
            pltpu.sync_copy(rows.at[pl.ds(0, rtail)],
                            u_sh.at[pl.ds(nrk * ch, rtail)])
        plsc.subcore_barrier()

        @pl.loop(0, nck)
        def _chunk(ci):
            start = (ci * NW + wid) * ch

            @pl.when(start < E)
            def _do():
                pltpu.sync_copy(src_hbm.at[pl.ds(start, ch)], srcb)
                pltpu.sync_copy(dst_hbm.at[pl.ds(start, ch)], dstb)
                for k in range(h_heads):
                    pltpu.sync_copy(ee_hbm.at[pl.ds(k * E + start, ch)],
                                    eeb.at[pl.ds(k * ch, ch)])
                pltpu.sync_copy(h_hbm.at[srcb], rows)

                @pl.loop(0, ch)
                def _scale(e):
                    ev = jnp.full((16,), 0, jnp.int32) + e
                    for k in range(h_heads):
                        s = plsc.load_gather(eeb, [ev + (k * ch)])
                        for f in range(feats // 16):
                            jj = k * (feats // 16) + f
                            rows[e, pl.ds(jj * 16, 16)] = (
                                rows[e, pl.ds(jj * 16, 16)] * s)

                pltpu.sync_copy(rows, u_sh.at[dstb], add=True)

        plsc.subcore_barrier()

        @pl.loop(0, nrk)
        def _st(bi):
            @pl.when(bi % NS == sid)
            def _():
                pltpu.sync_copy(u_sh.at[pl.ds(bi * ch, ch)],
                                u_hbm.at[cid, pl.ds(bi * ch, ch)])

        @pl.when(sid == 0)
        def _stt():
            pltpu.sync_copy(u_sh.at[pl.ds(nrk * ch, rtail)],
                            u_hbm.at[cid, pl.ds(nrk * ch, rtail)])

    return pass2


def _attn_mat(attn, in_dim):
    h, f = attn.shape
    mask = (jnp.arange(in_dim)[:, None] // f) == jnp.arange(h)[None, :]
    return jnp.where(mask, attn.reshape(-1)[:, None], 0.0).astype(_F32)


def kernel(feat, edge_index, W1, al1, ar1, b1, W2, al2, ar2, b2):
    src = edge_index[0].astype(jnp.int32)
    dst = edge_index[1].astype(jnp.int32)

    b1mat = jnp.concatenate(
        [_attn_mat(al1, IN_F), _attn_mat(ar1, IN_F)], axis=1)
    b2mat = jnp.concatenate(
        [_attn_mat(al2, NCLS), _attn_mat(ar2, NCLS)], axis=1)
    b2mat = jnp.concatenate(
        [b2mat, jnp.zeros((IN_F - NCLS, 2), _F32)], axis=0)
    krep = jnp.concatenate([jnp.eye(NCLS, dtype=_F32)] * (IN_F // NCLS),
                           axis=1)
    w2rep = jnp.dot(W2, krep)
    p1 = ((jnp.arange(IN_F)[None, :] // HID)
          == jnp.arange(HEADS)[:, None]).astype(_F32)
    p2 = jnp.ones((1, NCLS), _F32)

    _pass1_l1 = _make_pass1(HEADS)
    _pass1_l2 = _make_pass1(1)
    _pass2_l1 = _make_pass2(HEADS, HID)
    _pass2_l2 = _make_pass2(1, IN_F)

    h1, elr1 = _proj1(feat, W1, b1mat)
    ee1, esum1 = _pass1_l1(elr1.reshape(-1), src, dst)
    u1 = _pass2_l1(h1, src, dst, ee1)
    inv1 = _inv_reduce(esum1.reshape(NW, -1)).reshape(N, HEADS)
    h2, elr2 = _combine(u1, inv1, p1, b1.reshape(1, -1), w2rep, b2mat)
    ee2, esum2 = _pass1_l2(elr2.reshape(-1), src, dst)
    u2 = _pass2_l2(h2, src, dst, ee2)
    inv2 = _inv_reduce(esum2.reshape(NW, -1)).reshape(N, 1)
    out = _final(u2, inv2, p2, b2.reshape(1, -1))
    return out


# speedup vs baseline: 47.2917x; 1.5940x over previous
import dataclasses
import functools

import jax
import jax.numpy as jnp
from jax import lax
from jax.experimental import pallas as pl
from jax.experimental.pallas import tpu as pltpu
from jax.experimental.pallas import tpu_sc as plsc

N = 10000
IN_F = 128
HID = 32
HEADS = 4
NCLS = 16
E = 320000

NC = 2
NS = 16
NW = NC * NS
EPW = E // NW

_F32 = jnp.float32

_SC_CP = pltpu.CompilerParams()
if "needs_layout_passes" in pltpu.CompilerParams.__dataclass_fields__:
    _SC_CP = dataclasses.replace(_SC_CP, needs_layout_passes=False)


def _proj1_body(feat_ref, w_ref, b_ref, h_ref, elr_ref):
    h = jnp.dot(feat_ref[...], w_ref[...], preferred_element_type=_F32)
    h_ref[...] = h
    elr_ref[...] = jnp.dot(h, b_ref[...], preferred_element_type=_F32)


def _proj1(feat, w1, b1mat):
    blk = 1000
    return pl.pallas_call(
        _proj1_body,
        grid=(N // blk,),
        in_specs=[
            pl.BlockSpec((blk, IN_F), lambda i: (i, 0)),
            pl.BlockSpec((IN_F, IN_F), lambda i: (0, 0)),
            pl.BlockSpec((IN_F, 2 * HEADS), lambda i: (0, 0)),
        ],
        out_specs=[
            pl.BlockSpec((blk, IN_F), lambda i: (i, 0)),
            pl.BlockSpec((blk, 2 * HEADS), lambda i: (i, 0)),
        ],
        out_shape=[
            jax.ShapeDtypeStruct((N, IN_F), _F32),
            jax.ShapeDtypeStruct((N, 2 * HEADS), _F32),
        ],
    )(feat, w1, b1mat)


def _inv_body(es_ref, inv_ref):
    inv_ref[...] = 1.0 / (jnp.sum(es_ref[...], axis=0) + 1e-9)


def _inv_reduce(esum_parts):
    m = esum_parts.shape[1]
    return pl.pallas_call(
        _inv_body,
        in_specs=[pl.BlockSpec((NW, m), lambda: (0, 0))],
        out_specs=pl.BlockSpec((m,), lambda: (0,)),
        out_shape=jax.ShapeDtypeStruct((m,), _F32),
    )(esum_parts)


def _combine_body(u_ref, inv_ref, p_ref, b1_ref, w2_ref, b2m_ref,
                  h2_ref, elr_ref):
    inv_exp = jnp.dot(inv_ref[...], p_ref[...],
                      preferred_element_type=_F32)
    u = u_ref[0] + u_ref[1]
    o1 = jnp.maximum(u * inv_exp + b1_ref[...], 0.0)
    h2 = jnp.dot(o1, w2_ref[...], preferred_element_type=_F32)
    h2_ref[...] = h2
    elr_ref[...] = jnp.dot(h2, b2m_ref[...], preferred_element_type=_F32)


def _combine(u_parts, inv1, p1, b1flat, w2, b2mat):
    blk = 1000
    return pl.pallas_call(
        _combine_body,
        grid=(N // blk,),
        in_specs=[
            pl.BlockSpec((NC, blk, IN_F), lambda i: (0, i, 0)),
            pl.BlockSpec((blk, HEADS), lambda i: (i, 0)),
            pl.BlockSpec((HEADS, IN_F), lambda i: (0, 0)),
            pl.BlockSpec((1, IN_F), lambda i: (0, 0)),
            pl.BlockSpec((IN_F, IN_F), lambda i: (0, 0)),
            pl.BlockSpec((IN_F, 2), lambda i: (0, 0)),
        ],
        out_specs=[
            pl.BlockSpec((blk, IN_F), lambda i: (i, 0)),
            pl.BlockSpec((blk, 2), lambda i: (i, 0)),
        ],
        out_shape=[
            jax.ShapeDtypeStruct((N, IN_F), _F32),
            jax.ShapeDtypeStruct((N, 2), _F32),
        ],
    )(u_parts, inv1, p1, b1flat, w2, b2mat)


def _final_body(u_ref, inv_ref, p_ref, b2_ref, out_ref):
    inv_exp = jnp.dot(inv_ref[...], p_ref[...],
                      preferred_element_type=_F32)
    u = u_ref[0][:, :NCLS] + u_ref[1][:, :NCLS]
    out_ref[...] = u * inv_exp + b2_ref[...]


def _final(u_parts, inv2, p2, b2flat):
    blk = 1000
    return pl.pallas_call(
        _final_body,
        grid=(N // blk,),
        in_specs=[
            pl.BlockSpec((NC, blk, IN_F), lambda i: (0, i, 0)),
            pl.BlockSpec((blk, 1), lambda i: (i, 0)),
            pl.BlockSpec((1, NCLS), lambda i: (0, 0)),
            pl.BlockSpec((1, NCLS), lambda i: (0, 0)),
        ],
        out_specs=pl.BlockSpec((blk, NCLS), lambda i: (i, 0)),
        out_shape=jax.ShapeDtypeStruct((N, NCLS), _F32),
    )(u_parts, inv2, p2, b2flat)


def _make_pass1(h_heads):
    tbl_sz = 2 * h_heads * N
    acc_sz = h_heads * N
    ch = 640
    nck = -(-E // ch // NW)
    mesh = plsc.VectorSubcoreMesh(core_axis_name="c", subcore_axis_name="s",
                                  num_cores=NC, num_subcores=NS)

    @functools.partial(
        pl.kernel,
        out_type=(
            jax.ShapeDtypeStruct((h_heads * E,), _F32),
            jax.ShapeDtypeStruct((NW, 1, acc_sz), _F32),
        ),
        mesh=mesh,
        compiler_params=_SC_CP,
        scratch_types=[
            pltpu.VMEM((tbl_sz,), _F32),
            pltpu.VMEM((acc_sz,), _F32),
            pltpu.VMEM((ch,), jnp.int32),
            pltpu.VMEM((ch,), jnp.int32),
            pltpu.VMEM((h_heads * ch,), _F32),
        ],
    )
    def pass1(elr_hbm, src_hbm, dst_hbm, ee_hbm, esum_hbm,
              tbl, acc, srcb, dstb, eeb):
        cid = lax.axis_index("c")
        sid = lax.axis_index("s")
        wid = sid * NC + cid
        pltpu.sync_copy(elr_hbm, tbl)
        zeros = jnp.zeros((16,), _F32)

        @pl.loop(0, acc_sz, step=16)
        def _zero(i):
            acc[pl.ds(i, 16)] = zeros

        @pl.loop(0, nck)
        def _chunk(ci):
            start = (ci * NW + wid) * ch

            @pl.when(start < E)
            def _do():
                pltpu.sync_copy(src_hbm.at[pl.ds(start, ch)], srcb)
                pltpu.sync_copy(dst_hbm.at[pl.ds(start, ch)], dstb)

                @pl.loop(0, ch, step=16)
                def _grp(g):
                    sv = srcb[pl.ds(g, 16)]
                    dv = dstb[pl.ds(g, 16)]
                    sv2 = sv * (2 * h_heads)
                    dv2 = dv * (2 * h_heads)
                    dva = dv * h_heads
                    for k in range(h_heads):
                        el = plsc.load_gather(tbl, [sv2 + k])
                        er = plsc.load_gather(tbl, [dv2 + (h_heads + k)])
                        e = el + er
                        e = jnp.maximum(e, 0.2 * e)
                        ee = jnp.exp(e)
                        gv = jnp.arange(16, dtype=jnp.int32) + g
                        plsc.store_scatter(eeb, [gv * h_heads + k], ee)
                        plsc.addupdate_scatter(acc, [dva + k], ee)

                pltpu.sync_copy(eeb,
                                ee_hbm.at[pl.ds(start * h_heads,
                                                ch * h_heads)])

        pltpu.sync_copy(acc, esum_hbm.at[wid, 0])

    return pass1


def _make_pass2(h_heads, feats):
    d_model = h_heads * feats
    ch = 128
    nck = -(-E // ch // NW)
    nck2 = nck + (nck % 2)
    nrk = N // ch
    rtail = N - nrk * ch
    mesh = plsc.VectorSubcoreMesh(core_axis_name="c", subcore_axis_name="s",
                                  num_cores=NC, num_subcores=NS)

    @functools.partial(
        pl.kernel,
        out_type=jax.ShapeDtypeStruct((NC, N, d_model), _F32),
        mesh=mesh,
        compiler_params=_SC_CP,
        scratch_types=[
            pltpu.VMEM((2, ch), jnp.int32),
            pltpu.VMEM((2, ch), jnp.int32),
            pltpu.VMEM((2, h_heads * ch), _F32),
            pltpu.VMEM((2, ch, d_model), _F32),
            pltpu.VMEM_SHARED((N, d_model), _F32),
            pltpu.SemaphoreType.DMA,
            pltpu.SemaphoreType.DMA,
            pltpu.SemaphoreType.DMA,
            pltpu.SemaphoreType.DMA,
        ],
    )
    def pass2(h_hbm, src_hbm, dst_hbm, ee_hbm, u_hbm,
              srcb, dstb, eeb, rows, u_sh,
              sem_in0, sem_in1, sem_g0, sem_g1):
        sem_in = (sem_in0, sem_in1)
        sem_g = (sem_g0, sem_g1)
        cid = lax.axis_index("c")
        sid = lax.axis_index("s")
        wid = sid * NC + cid
        zeros = jnp.zeros((16,), _F32)

        def cstart(ci):
            return (ci * NW + wid) * ch

        def in_copies(b, ci):
            start = cstart(ci)
            return (
                pltpu.make_async_copy(src_hbm.at[pl.ds(start, ch)],
                                      srcb.at[b], sem_in[b]),
                pltpu.make_async_copy(dst_hbm.at[pl.ds(start, ch)],
                                      dstb.at[b], sem_in[b]),
                pltpu.make_async_copy(
                    ee_hbm.at[pl.ds(start * h_heads, ch * h_heads)],
                    eeb.at[b], sem_in[b]),
            )

        def gather_copy(b):
            return pltpu.make_async_copy(h_hbm.at[srcb.at[b]],
                                         rows.at[b], sem_g[b])

        @pl.loop(0, ch)
        def _zr(r):
            for jj in range(d_model // 16):
                rows[0, r, pl.ds(jj * 16, 16)] = zeros

        @pl.loop(0, nrk)
        def _zs(bi):
            @pl.when(bi % NS == sid)
            def _():
                pltpu.sync_copy(rows.at[0], u_sh.at[pl.ds(bi * ch, ch)])

        @pl.when(sid == 0)
        def _zt():
            pltpu.sync_copy(rows.at[0].at[pl.ds(0, rtail)],
                            u_sh.at[pl.ds(nrk * ch, rtail)])
        plsc.subcore_barrier()

        @pl.when(cstart(0) < E)
        def _p0():
            for c in in_copies(0, 0):
                c.start()

        @pl.when(cstart(1) < E)
        def _p1():
            for c in in_copies(1, 1):
                c.start()

        @pl.when(cstart(0) < E)
        def _p2():
            for c in in_copies(0, 0):
                c.wait()
            gather_copy(0).start()

        @pl.loop(0, nck2, step=2)
        def _chunk(ci0):
            for b in (0, 1):
                ci = ci0 + b
                ob = 1 - b

                @pl.when(cstart(ci) < E)
                def _wg():
                    gather_copy(b).wait()

                @pl.when(cstart(ci + 1) < E)
                def _ng():
                    for c in in_copies(ob, ci + 1):
                        c.wait()
                    gather_copy(ob).start()

                @pl.when(cstart(ci) < E)
                def _work():
                    @pl.loop(0, ch)
                    def _scale(e):
                        ev = jnp.full((16,), 0, jnp.int32) + (e * h_heads)
                        bv = jnp.full((16,), b, jnp.int32)
                        for k in range(h_heads):
                            sk = plsc.load_gather(eeb, [bv, ev + k])
                            for f in range(feats // 16):
                                jj = k * (feats // 16) + f
                                rows[b, e, pl.ds(jj * 16, 16)] = (
                                    rows[b, e, pl.ds(jj * 16, 16)] * sk)

                    pltpu.sync_copy(rows.at[b], u_sh.at[dstb.at[b]], add=True)

                    @pl.when(cstart(ci + 2) < E)
                    def _ni():
                        for c in in_copies(b, ci + 2):
                            c.start()

        plsc.subcore_barrier()

        @pl.loop(0, nrk)
        def _st(bi):
            @pl.when(bi % NS == sid)
            def _():
                pltpu.sync_copy(u_sh.at[pl.ds(bi * ch, ch)],
                                u_hbm.at[cid, pl.ds(bi * ch, ch)])

        @pl.when(sid == 0)
        def _stt():
            pltpu.sync_copy(u_sh.at[pl.ds(nrk * ch, rtail)],
                            u_hbm.at[cid, pl.ds(nrk * ch, rtail)])

    return pass2


def _attn_mat(attn, in_dim):
    h, f = attn.shape
    mask = (jnp.arange(in_dim)[:, None] // f) == jnp.arange(h)[None, :]
    return jnp.where(mask, attn.reshape(-1)[:, None], 0.0).astype(_F32)


def kernel(feat, edge_index, W1, al1, ar1, b1, W2, al2, ar2, b2):
    src = edge_index[0].astype(jnp.int32)
    dst = edge_index[1].astype(jnp.int32)

    b1mat = jnp.concatenate(
        [_attn_mat(al1, IN_F), _attn_mat(ar1, IN_F)], axis=1)
    b2mat = jnp.concatenate(
        [_attn_mat(al2, NCLS), _attn_mat(ar2, NCLS)], axis=1)
    b2mat = jnp.concatenate(
        [b2mat, jnp.zeros((IN_F - NCLS, 2), _F32)], axis=0)
    krep = jnp.concatenate([jnp.eye(NCLS, dtype=_F32)] * (IN_F // NCLS),
                           axis=1)
    w2rep = jnp.dot(W2, krep)
    p1 = ((jnp.arange(IN_F)[None, :] // HID)
          == jnp.arange(HEADS)[:, None]).astype(_F32)
    p2 = jnp.ones((1, NCLS), _F32)

    _pass1_l1 = _make_pass1(HEADS)
    _pass1_l2 = _make_pass1(1)
    _pass2_l1 = _make_pass2(HEADS, HID)
    _pass2_l2 = _make_pass2(1, IN_F)

    h1, elr1 = _proj1(feat, W1, b1mat)
    ee1, esum1 = _pass1_l1(elr1.reshape(-1), src, dst)
    u1 = _pass2_l1(h1, src, dst, ee1)
    inv1 = _inv_reduce(esum1.reshape(NW, -1)).reshape(N, HEADS)
    h2, elr2 = _combine(u1, inv1, p1, b1.reshape(1, -1), w2rep, b2mat)
    ee2, esum2 = _pass1_l2(elr2.reshape(-1), src, dst)
    u2 = _pass2_l2(h2, src, dst, ee2)
    inv2 = _inv_reduce(esum2.reshape(NW, -1)).reshape(N, 1)
    out = _final(u2, inv2, p2, b2.reshape(1, -1))
    return out
